# baseline (speedup 1.0000x reference)
import functools

import jax
import jax.numpy as jnp
from jax import lax
from jax.experimental import pallas as pl
from jax.experimental.pallas import tpu as pltpu


def kernel(x, perm):
    n_rows, n_cols = x.shape
    B = 256

    n_blk = n_cols // 128

    def body(x_ref, o_ref):
        ridx = lax.broadcasted_iota(jnp.int32, (B, 128), 1)
        for j in range(n_blk):
            src = x_ref[:, (n_blk - 1 - j) * 128 : (n_blk - j) * 128]
            o_ref[:, j * 128 : (j + 1) * 128] = jnp.take_along_axis(
                src, 127 - ridx, axis=1
            )

    out = pl.pallas_call(
        body,
        grid=(n_rows // B,),
        in_specs=[pl.BlockSpec((B, n_cols), lambda i: (i, 0))],
        out_specs=pl.BlockSpec((B, n_cols), lambda i: (i, 0)),
        out_shape=jax.ShapeDtypeStruct((n_rows, n_cols), jnp.float32),
    )(x)
    return (out, 0.0)
